# Initial kernel scaffold; baseline (speedup 1.0000x reference)
import jax
import jax.numpy as jnp
from jax.experimental import pallas as pl


def kernel(x, emb_table, W1, b1, W2, b2):
    raise NotImplementedError("write your pallas kernel here")


# speedup vs baseline: 11.9382x; 11.9382x over previous
import functools

import jax
import jax.numpy as jnp
from jax import lax
from jax.experimental import pallas as pl
from jax.experimental.pallas import tpu as pltpu
from jax.experimental.pallas import tpu_sc as plsc

VOCAB = 1000000
EMBED = 32
NUM_CLASSES = 100
BATCH = 16384
SEQ = 200

NC = 2
NS = 16
NW = NC * NS
ROWS_W = BATCH // NW
GCH = 40
CPR = SEQ // GCH
CROWS_W = ROWS_W * CPR
PCHUNK = 64

_HALF = EMBED // 2


def _pool_body(xi_hbm, tab_hbm, out_hbm, idx_v, rows_v, pool_v, gsem, osem):
    wid = lax.axis_index("s") * NC + lax.axis_index("c")
    cbase = pl.multiple_of(wid * CROWS_W, CROWS_W)
    rbase = pl.multiple_of(wid * ROWS_W, ROWS_W)

    pltpu.sync_copy(xi_hbm.at[pl.ds(cbase, CROWS_W)], idx_v)

    for b in range(CPR):
        pltpu.async_copy(tab_hbm.at[idx_v.at[b]], rows_v.at[b], gsem.at[b])

    inv = 1.0 / SEQ

    @pl.loop(0, ROWS_W)
    def _row(r):
        sums0 = []
        sums1 = []
        for b in range(CPR):
            pltpu.make_async_copy(
                tab_hbm.at[idx_v.at[r * CPR + b]], rows_v.at[b], gsem.at[b]
            ).wait()
            a0 = rows_v[b, 0, 0:_HALF]
            a1 = rows_v[b, 0, _HALF:EMBED]
            c0 = rows_v[b, 1, 0:_HALF]
            c1 = rows_v[b, 1, _HALF:EMBED]
            for l in range(2, GCH, 2):
                a0 = a0 + rows_v[b, l, 0:_HALF]
                a1 = a1 + rows_v[b, l, _HALF:EMBED]
                c0 = c0 + rows_v[b, l + 1, 0:_HALF]
                c1 = c1 + rows_v[b, l + 1, _HALF:EMBED]
            sums0.append(a0 + c0)
            sums1.append(a1 + c1)

            @pl.when(r + 1 < ROWS_W)
            def _():
                pltpu.async_copy(
                    tab_hbm.at[idx_v.at[(r + 1) * CPR + b]],
                    rows_v.at[b],
                    gsem.at[b],
                )

        s0 = ((sums0[0] + sums0[1]) + (sums0[2] + sums0[3]) + sums0[4]) * inv
        s1 = ((sums1[0] + sums1[1]) + (sums1[2] + sums1[3]) + sums1[4]) * inv
        pbuf = (r // PCHUNK) % 2
        slot = r % PCHUNK
        pool_v[pbuf, slot, 0:_HALF] = s0
        pool_v[pbuf, slot, _HALF:EMBED] = s1

        @pl.when(slot == PCHUNK - 1)
        def _flush():
            @pl.when(r >= 2 * PCHUNK - 1)
            def _():
                pltpu.make_async_copy(
                    pool_v.at[0], out_hbm.at[pl.ds(rbase, PCHUNK)], osem
                ).wait()

            pltpu.async_copy(
                pool_v.at[pbuf],
                out_hbm.at[pl.ds(pl.multiple_of(rbase + r + 1 - PCHUNK, PCHUNK), PCHUNK)],
                osem,
            )

    pltpu.make_async_copy(
        pool_v.at[0], out_hbm.at[pl.ds(rbase, PCHUNK)], osem
    ).wait()


_pool = functools.partial(
    pl.kernel,
    out_type=jax.ShapeDtypeStruct((BATCH, EMBED), jnp.float32),
    mesh=plsc.VectorSubcoreMesh(
        core_axis_name="c", subcore_axis_name="s", num_cores=NC, num_subcores=NS
    ),
    scratch_types=[
        pltpu.VMEM((CROWS_W, GCH), jnp.int32),
        pltpu.VMEM((CPR, GCH, EMBED), jnp.float32),
        pltpu.VMEM((2, PCHUNK, EMBED), jnp.float32),
        pltpu.SemaphoreType.DMA((CPR,)),
        pltpu.SemaphoreType.DMA,
    ],
    compiler_params=pltpu.CompilerParams(use_tc_tiling_on_sc=False),
)(_pool_body)


def _mlp_body(p_ref, w1_ref, b1_ref, w2_ref, b2_ref, o_ref):
    h = jnp.dot(p_ref[...], w1_ref[...], preferred_element_type=jnp.float32)
    h = jnp.maximum(h + b1_ref[...], 0.0)
    o_ref[...] = (
        jnp.dot(h, w2_ref[...], preferred_element_type=jnp.float32) + b2_ref[...]
    )


_MB = 2048

_mlp = pl.pallas_call(
    _mlp_body,
    grid=(BATCH // _MB,),
    in_specs=[
        pl.BlockSpec((_MB, EMBED), lambda i: (i, 0)),
        pl.BlockSpec((EMBED, 2 * EMBED), lambda i: (0, 0)),
        pl.BlockSpec((1, 2 * EMBED), lambda i: (0, 0)),
        pl.BlockSpec((2 * EMBED, NUM_CLASSES), lambda i: (0, 0)),
        pl.BlockSpec((1, NUM_CLASSES), lambda i: (0, 0)),
    ],
    out_specs=pl.BlockSpec((_MB, NUM_CLASSES), lambda i: (i, 0)),
    out_shape=jax.ShapeDtypeStruct((BATCH, NUM_CLASSES), jnp.float32),
)


def kernel(x, emb_table, W1, b1, W2, b2):
    xi = x.reshape(BATCH * CPR, GCH)
    pooled = _pool(xi, emb_table)
    return _mlp(
        pooled,
        W1.T,
        b1.reshape(1, 2 * EMBED),
        W2.T,
        b2.reshape(1, NUM_CLASSES),
    )


# speedup vs baseline: 11.9465x; 1.0007x over previous
import functools

import jax
import jax.numpy as jnp
from jax import lax
from jax.experimental import pallas as pl
from jax.experimental.pallas import tpu as pltpu
from jax.experimental.pallas import tpu_sc as plsc

VOCAB = 1000000
EMBED = 32
NUM_CLASSES = 100
BATCH = 16384
SEQ = 200

NC = 2
NS = 16
NW = NC * NS
ROWS_W = BATCH // NW
GCH = 40
CPR = SEQ // GCH
CROWS_W = ROWS_W * CPR
PCHUNK = 64

_HALF = EMBED // 2


def _pool_body(xi_hbm, tab_hbm, out_hbm, idx_v, rows_v, pool_v, gsem, osem):
    wid = lax.axis_index("s") * NC + lax.axis_index("c")
    rbase = pl.multiple_of(wid * ROWS_W, ROWS_W)

    pltpu.sync_copy(xi_hbm.at[pl.ds(rbase, ROWS_W)], idx_v)

    for b in range(CPR):
        pltpu.async_copy(
            tab_hbm.at[idx_v.at[0, pl.ds(b * GCH, GCH)]], rows_v.at[b], gsem.at[b]
        )

    inv = 1.0 / SEQ

    @pl.loop(0, ROWS_W)
    def _row(r):
        sums0 = []
        sums1 = []
        for b in range(CPR):
            pltpu.make_async_copy(
                tab_hbm.at[idx_v.at[r, pl.ds(b * GCH, GCH)]],
                rows_v.at[b],
                gsem.at[b],
            ).wait()
            a0 = rows_v[b, 0, 0:_HALF]
            a1 = rows_v[b, 0, _HALF:EMBED]
            c0 = rows_v[b, 1, 0:_HALF]
            c1 = rows_v[b, 1, _HALF:EMBED]
            for l in range(2, GCH, 2):
                a0 = a0 + rows_v[b, l, 0:_HALF]
                a1 = a1 + rows_v[b, l, _HALF:EMBED]
                c0 = c0 + rows_v[b, l + 1, 0:_HALF]
                c1 = c1 + rows_v[b, l + 1, _HALF:EMBED]
            sums0.append(a0 + c0)
            sums1.append(a1 + c1)

            @pl.when(r + 1 < ROWS_W)
            def _():
                pltpu.async_copy(
                    tab_hbm.at[idx_v.at[r + 1, pl.ds(b * GCH, GCH)]],
                    rows_v.at[b],
                    gsem.at[b],
                )

        s0 = ((sums0[0] + sums0[1]) + (sums0[2] + sums0[3]) + sums0[4]) * inv
        s1 = ((sums1[0] + sums1[1]) + (sums1[2] + sums1[3]) + sums1[4]) * inv
        pbuf = (r // PCHUNK) % 2
        slot = r % PCHUNK
        pool_v[pbuf, slot, 0:_HALF] = s0
        pool_v[pbuf, slot, _HALF:EMBED] = s1

        @pl.when(slot == PCHUNK - 1)
        def _flush():
            @pl.when(r >= 2 * PCHUNK - 1)
            def _():
                pltpu.make_async_copy(
                    pool_v.at[0], out_hbm.at[pl.ds(rbase, PCHUNK)], osem
                ).wait()

            pltpu.async_copy(
                pool_v.at[pbuf],
                out_hbm.at[pl.ds(pl.multiple_of(rbase + r + 1 - PCHUNK, PCHUNK), PCHUNK)],
                osem,
            )

    pltpu.make_async_copy(
        pool_v.at[0], out_hbm.at[pl.ds(rbase, PCHUNK)], osem
    ).wait()


_pool = functools.partial(
    pl.kernel,
    out_type=jax.ShapeDtypeStruct((BATCH, EMBED), jnp.float32),
    mesh=plsc.VectorSubcoreMesh(
        core_axis_name="c", subcore_axis_name="s", num_cores=NC, num_subcores=NS
    ),
    scratch_types=[
        pltpu.VMEM((ROWS_W, SEQ), jnp.int32),
        pltpu.VMEM((CPR, GCH, EMBED), jnp.float32),
        pltpu.VMEM((2, PCHUNK, EMBED), jnp.float32),
        pltpu.SemaphoreType.DMA((CPR,)),
        pltpu.SemaphoreType.DMA,
    ],
    compiler_params=pltpu.CompilerParams(use_tc_tiling_on_sc=False),
)(_pool_body)


def _mlp_body(p_ref, w1_ref, b1_ref, w2_ref, b2_ref, o_ref):
    h = jnp.dot(p_ref[...], w1_ref[...], preferred_element_type=jnp.float32)
    h = jnp.maximum(h + b1_ref[...], 0.0)
    o_ref[...] = (
        jnp.dot(h, w2_ref[...], preferred_element_type=jnp.float32) + b2_ref[...]
    )


_MB = 2048

_mlp = pl.pallas_call(
    _mlp_body,
    grid=(BATCH // _MB,),
    in_specs=[
        pl.BlockSpec((_MB, EMBED), lambda i: (i, 0)),
        pl.BlockSpec((EMBED, 2 * EMBED), lambda i: (0, 0)),
        pl.BlockSpec((1, 2 * EMBED), lambda i: (0, 0)),
        pl.BlockSpec((2 * EMBED, NUM_CLASSES), lambda i: (0, 0)),
        pl.BlockSpec((1, NUM_CLASSES), lambda i: (0, 0)),
    ],
    out_specs=pl.BlockSpec((_MB, NUM_CLASSES), lambda i: (i, 0)),
    out_shape=jax.ShapeDtypeStruct((BATCH, NUM_CLASSES), jnp.float32),
)


def kernel(x, emb_table, W1, b1, W2, b2):
    pooled = _pool(x, emb_table)
    return _mlp(
        pooled,
        W1.T,
        b1.reshape(1, 2 * EMBED),
        W2.T,
        b2.reshape(1, NUM_CLASSES),
    )


# speedup vs baseline: 14.4812x; 1.2122x over previous
import functools

import jax
import jax.numpy as jnp
from jax import lax
from jax.experimental import pallas as pl
from jax.experimental.pallas import tpu as pltpu
from jax.experimental.pallas import tpu_sc as plsc

VOCAB = 1000000
EMBED = 32
NUM_CLASSES = 100
BATCH = 16384
SEQ = 200

NC = 2
NS = 16
NW = NC * NS
ROWS_W = BATCH // NW
GCH = 40
CPR = SEQ // GCH
CROWS_W = ROWS_W * CPR
PCHUNK = 64

_HALF = EMBED // 2


GA = 104
GB = SEQ - GA


def _pool_body(xi_hbm, tab_hbm, out_hbm, idx_v, rowsa_v, rowsb_v, pool_v, gsem, osem):
    wid = lax.axis_index("s") * NC + lax.axis_index("c")
    rbase = pl.multiple_of(wid * ROWS_W, ROWS_W)

    pltpu.sync_copy(xi_hbm.at[pl.ds(rbase, ROWS_W)], idx_v)

    def issue(r, s):
        pltpu.async_copy(
            tab_hbm.at[idx_v.at[r, pl.ds(0, GA)]], rowsa_v.at[s], gsem.at[2 * s]
        )
        pltpu.async_copy(
            tab_hbm.at[idx_v.at[r, pl.ds(GA, GB)]], rowsb_v.at[s], gsem.at[2 * s + 1]
        )

    issue(0, 0)
    issue(1, 1)

    inv = 1.0 / SEQ

    @pl.loop(0, ROWS_W, step=2)
    def _row(r0):
        for s in range(2):
            r = r0 + s
            pltpu.make_async_copy(
                tab_hbm.at[idx_v.at[r, pl.ds(0, GA)]], rowsa_v.at[s], gsem.at[2 * s]
            ).wait()
            pltpu.make_async_copy(
                tab_hbm.at[idx_v.at[r, pl.ds(GA, GB)]], rowsb_v.at[s], gsem.at[2 * s + 1]
            ).wait()

            a0 = rowsa_v[s, 0, 0:_HALF]
            a1 = rowsa_v[s, 0, _HALF:EMBED]
            c0 = rowsa_v[s, 1, 0:_HALF]
            c1 = rowsa_v[s, 1, _HALF:EMBED]
            for l in range(2, GA, 2):
                a0 = a0 + rowsa_v[s, l, 0:_HALF]
                a1 = a1 + rowsa_v[s, l, _HALF:EMBED]
                c0 = c0 + rowsa_v[s, l + 1, 0:_HALF]
                c1 = c1 + rowsa_v[s, l + 1, _HALF:EMBED]
            for l in range(0, GB, 2):
                a0 = a0 + rowsb_v[s, l, 0:_HALF]
                a1 = a1 + rowsb_v[s, l, _HALF:EMBED]
                c0 = c0 + rowsb_v[s, l + 1, 0:_HALF]
                c1 = c1 + rowsb_v[s, l + 1, _HALF:EMBED]

            @pl.when(r + 2 < ROWS_W)
            def _():
                issue(r + 2, s)

            s0 = (a0 + c0) * inv
            s1 = (a1 + c1) * inv
            pbuf = (r // PCHUNK) % 2
            slot = r % PCHUNK
            pool_v[pbuf, slot, 0:_HALF] = s0
            pool_v[pbuf, slot, _HALF:EMBED] = s1

            @pl.when(slot == PCHUNK - 1)
            def _flush():
                @pl.when(r >= 2 * PCHUNK - 1)
                def _():
                    pltpu.make_async_copy(
                        pool_v.at[0], out_hbm.at[pl.ds(rbase, PCHUNK)], osem
                    ).wait()

                pltpu.async_copy(
                    pool_v.at[pbuf],
                    out_hbm.at[
                        pl.ds(pl.multiple_of(rbase + r + 1 - PCHUNK, PCHUNK), PCHUNK)
                    ],
                    osem,
                )

    pltpu.make_async_copy(
        pool_v.at[0], out_hbm.at[pl.ds(rbase, PCHUNK)], osem
    ).wait()


_pool = functools.partial(
    pl.kernel,
    out_type=jax.ShapeDtypeStruct((BATCH, EMBED), jnp.float32),
    mesh=plsc.VectorSubcoreMesh(
        core_axis_name="c", subcore_axis_name="s", num_cores=NC, num_subcores=NS
    ),
    scratch_types=[
        pltpu.VMEM((ROWS_W, SEQ), jnp.int32),
        pltpu.VMEM((2, GA, EMBED), jnp.float32),
        pltpu.VMEM((2, GB, EMBED), jnp.float32),
        pltpu.VMEM((2, PCHUNK, EMBED), jnp.float32),
        pltpu.SemaphoreType.DMA((4,)),
        pltpu.SemaphoreType.DMA,
    ],
    compiler_params=pltpu.CompilerParams(use_tc_tiling_on_sc=False),
)(_pool_body)


_TW = 8192
_TGRID = (VOCAB + _TW - 1) // _TW


def _tab_tr_body(in_ref, out_ref):
    t = in_ref[...].T.reshape(_TW // 4, 4, EMBED)
    out_ref[...] = jnp.concatenate([t[:, a, :] for a in range(4)], axis=1)


_tab_tr = pl.pallas_call(
    _tab_tr_body,
    grid=(_TGRID,),
    in_specs=[pl.BlockSpec((EMBED, _TW), lambda i: (0, i))],
    out_specs=pl.BlockSpec((_TW // 4, 4 * EMBED), lambda i: (i, 0)),
    out_shape=jax.ShapeDtypeStruct((VOCAB // 4, 4 * EMBED), jnp.float32),
)


def _mlp_body(p_ref, w1_ref, b1_ref, w2_ref, b2_ref, o_ref):
    h = jnp.dot(p_ref[...], w1_ref[...], preferred_element_type=jnp.float32)
    h = jnp.maximum(h + b1_ref[...], 0.0)
    o_ref[...] = (
        jnp.dot(h, w2_ref[...], preferred_element_type=jnp.float32) + b2_ref[...]
    )


_MB = 2048

_mlp = pl.pallas_call(
    _mlp_body,
    grid=(BATCH // _MB,),
    in_specs=[
        pl.BlockSpec((_MB, EMBED), lambda i: (i, 0)),
        pl.BlockSpec((EMBED, 2 * EMBED), lambda i: (0, 0)),
        pl.BlockSpec((1, 2 * EMBED), lambda i: (0, 0)),
        pl.BlockSpec((2 * EMBED, NUM_CLASSES), lambda i: (0, 0)),
        pl.BlockSpec((1, NUM_CLASSES), lambda i: (0, 0)),
    ],
    out_specs=pl.BlockSpec((_MB, NUM_CLASSES), lambda i: (i, 0)),
    out_shape=jax.ShapeDtypeStruct((BATCH, NUM_CLASSES), jnp.float32),
)


def kernel(x, emb_table, W1, b1, W2, b2):
    tab = _tab_tr(emb_table.T).reshape(VOCAB, EMBED)
    pooled = _pool(x, tab)
    return _mlp(
        pooled,
        W1.T,
        b1.reshape(1, 2 * EMBED),
        W2.T,
        b2.reshape(1, NUM_CLASSES),
    )
